# baseline (speedup 1.0000x reference)
import functools

import jax
import jax.numpy as jnp
from jax import lax
from jax.experimental import pallas as pl
from jax.experimental.pallas import tpu as pltpu
from jax.experimental.pallas import tpu_sc as plsc

EMBED_DIM = 32
BATCH = 4096
HIST_LEN = 200

_info = plsc.get_sparse_core_info()
_NC, _NS = _info.num_cores, _info.num_subcores
_NW = _NC * _NS
_BB = BATCH // _NW
_H = 8
_NCH = HIST_LEN // _H
_DH = EMBED_DIM // 8
_SP = 129

_mesh = plsc.VectorSubcoreMesh(core_axis_name="c", subcore_axis_name="s")


@functools.partial(
    pl.kernel,
    mesh=_mesh,
    out_type=jax.ShapeDtypeStruct((HIST_LEN, _DH, _NW, 8, 128), jnp.float32),
    scratch_types=[
        pltpu.VMEM((HIST_LEN, _BB), jnp.int32),
        pltpu.VMEM((2 * _H * _BB, EMBED_DIM), jnp.float32),
        pltpu.VMEM((_H * _DH, 8, _SP), jnp.float32),
        pltpu.SemaphoreType.DMA,
        pltpu.SemaphoreType.DMA,
    ],
    compiler_params=pltpu.CompilerParams(
        use_tc_tiling_on_sc=False, needs_layout_passes=False),
)
def _sc_embed(xt_hbm, table_hbm, out_hbm, idx_v, rows_v, stage_v, gsem, osem):
    wid = lax.axis_index("s") * _NC + lax.axis_index("c")
    b0 = wid * _BB

    pltpu.sync_copy(xt_hbm.at[:, pl.ds(b0, _BB)], idx_v)

    def gather_chunk(c, p):
        for hh in range(_H):
            pltpu.async_copy(
                table_hbm.at[idx_v.at[c * _H + hh]],
                rows_v.at[pl.ds((p * _H + hh) * _BB, _BB)],
                gsem,
            )

    lane = lax.iota(jnp.int32, 16)
    dl_v = lane & 7
    dh_half = lane >> 3

    def chunk_body(c, carry):
        p = lax.rem(c, 2)
        for hh in range(_H):
            pltpu.make_async_copy(
                table_hbm.at[idx_v.at[0]], rows_v.at[pl.ds(0, _BB)], gsem
            ).wait()

        @pl.when(c + 1 < _NCH)
        def _():
            gather_chunk(c + 1, 1 - p)

        @pl.when(c > 0)
        def _():
            for _ in range(_DH):
                pltpu.make_async_copy(
                    stage_v.at[pl.ds(0, _H), :, pl.ds(0, 128)],
                    out_hbm.at[pl.ds(0, _H), 0, 0], osem
                ).wait()

        @plsc.parallel_loop(0, _H, step=1, unroll=1)
        def _row_body(hh):
            h = c * _H + hh
            base = (p * _H + hh) * _BB
            blk0 = dh_half * _H + jnp.full((16,), hh, jnp.int32)
            blk1 = blk0 + 2 * _H

            @plsc.parallel_loop(0, _BB, step=8, unroll=2)
            def _tok_block(t):
                colv_b = jnp.full((16,), t, jnp.int32)
                for ti in range(8):
                    colv = colv_b + ti
                    v0 = rows_v[base + t + ti, pl.ds(0, 16)]
                    plsc.store_scatter(stage_v, [blk0, dl_v, colv], v0)
                    v1 = rows_v[base + t + ti, pl.ds(16, 16)]
                    plsc.store_scatter(stage_v, [blk1, dl_v, colv], v1)

        for dh in range(_DH):
            pltpu.async_copy(
                stage_v.at[pl.ds(dh * _H, _H), :, pl.ds(0, 128)],
                out_hbm.at[pl.ds(c * _H, _H), dh, wid], osem)
        return carry

    gather_chunk(0, 0)
    lax.fori_loop(0, _NCH, chunk_body, 0)
    for _ in range(_DH):
        pltpu.make_async_copy(
            stage_v.at[pl.ds(0, _H), :, pl.ds(0, 128)],
            out_hbm.at[pl.ds(0, _H), 0, 0], osem
        ).wait()


def kernel(x, table):
    xt = x.astype(jnp.int32).T
    out5 = _sc_embed(xt, table)
    return out5.transpose(2, 4, 0, 1, 3).reshape(BATCH, HIST_LEN, EMBED_DIM)


# speedup vs baseline: 1.0125x; 1.0125x over previous
import functools

import jax
import jax.numpy as jnp
from jax import lax
from jax.experimental import pallas as pl
from jax.experimental.pallas import tpu as pltpu
from jax.experimental.pallas import tpu_sc as plsc

EMBED_DIM = 32
BATCH = 4096
HIST_LEN = 200

_info = plsc.get_sparse_core_info()
_NC, _NS = _info.num_cores, _info.num_subcores
_NW = _NC * _NS
_BB = BATCH // _NW
_H = 8
_NCH = HIST_LEN // _H
_DH = EMBED_DIM // 8
_SP = 129

_mesh = plsc.VectorSubcoreMesh(core_axis_name="c", subcore_axis_name="s")


@functools.partial(
    pl.kernel,
    mesh=_mesh,
    out_type=jax.ShapeDtypeStruct((HIST_LEN, _DH, _NW, 8, 128), jnp.float32),
    scratch_types=[
        pltpu.VMEM((HIST_LEN, _BB), jnp.int32),
        pltpu.VMEM((2 * _H * _BB, EMBED_DIM), jnp.float32),
        pltpu.VMEM((_H * _DH, 8, _SP), jnp.float32),
        pltpu.SemaphoreType.DMA,
        pltpu.SemaphoreType.DMA,
    ],
    compiler_params=pltpu.CompilerParams(
        use_tc_tiling_on_sc=False, needs_layout_passes=False),
)
def _sc_embed(xt_hbm, table_hbm, out_hbm, idx_v, rows_v, stage_v, gsem, osem):
    wid = lax.axis_index("s") * _NC + lax.axis_index("c")
    b0 = wid * _BB

    pltpu.sync_copy(xt_hbm.at[:, pl.ds(b0, _BB)], idx_v)

    def gather_chunk(c, p):
        for hh in range(_H):
            pltpu.async_copy(
                table_hbm.at[idx_v.at[c * _H + hh]],
                rows_v.at[pl.ds((p * _H + hh) * _BB, _BB)],
                gsem,
            )

    lane = lax.iota(jnp.int32, 16)
    dl_v = lane & 7
    dh_half = lane >> 3

    def chunk_body(c, carry):
        p = lax.rem(c, 2)
        for hh in range(_H):
            pltpu.make_async_copy(
                table_hbm.at[idx_v.at[0]], rows_v.at[pl.ds(0, _BB)], gsem
            ).wait()

        @pl.when(c + 1 < _NCH)
        def _():
            gather_chunk(c + 1, 1 - p)

        @pl.when(c > 0)
        def _():
            for _ in range(_H * _DH):
                pltpu.make_async_copy(
                    stage_v.at[0, :, pl.ds(0, 128)], out_hbm.at[0, 0, 0], osem
                ).wait()

        @plsc.parallel_loop(0, _H, step=1, unroll=1)
        def _row_body(hh):
            h = c * _H + hh
            base = (p * _H + hh) * _BB
            blk0 = jnp.full((16,), hh * _DH, jnp.int32) + dh_half
            blk1 = blk0 + 2

            @plsc.parallel_loop(0, _BB, step=8, unroll=2)
            def _tok_block(t):
                colv_b = jnp.full((16,), t, jnp.int32)
                for ti in range(8):
                    colv = colv_b + ti
                    v0 = rows_v[base + t + ti, pl.ds(0, 16)]
                    plsc.store_scatter(stage_v, [blk0, dl_v, colv], v0)
                    v1 = rows_v[base + t + ti, pl.ds(16, 16)]
                    plsc.store_scatter(stage_v, [blk1, dl_v, colv], v1)

            for dh in range(_DH):
                blk = hh * _DH + dh
                pltpu.async_copy(
                    stage_v.at[blk, :, pl.ds(0, 128)],
                    out_hbm.at[h, dh, wid], osem)
        return carry

    gather_chunk(0, 0)
    lax.fori_loop(0, _NCH, chunk_body, 0)
    for _ in range(_H * _DH):
        pltpu.make_async_copy(
            stage_v.at[0, :, pl.ds(0, 128)], out_hbm.at[0, 0, 0], osem
        ).wait()


def kernel(x, table):
    xt = x.astype(jnp.int32).T
    out5 = _sc_embed(xt, table)
    return out5.transpose(2, 4, 0, 1, 3).reshape(BATCH, HIST_LEN, EMBED_DIM)
